# baseline (speedup 1.0000x reference)
import functools

import jax
import jax.numpy as jnp
from jax import lax
from jax.experimental import pallas as pl
from jax.experimental.pallas import tpu as pltpu
from jax.experimental.pallas import tpu_sc as plsc

_VOCAB = 1000000
_DIM = 64
_NEG = 5
_B = 16384
_CTX = 20
_TN = _NEG + 1

_NC, _NS, _L = 2, 16, 16
_NW = _NC * _NS
_BPW = _B // _NW
_CB = 64
_NCHUNK = _BPW // _CB
_CTX_IDX_ROWS = _CB * _CTX // 128
_TN_IDX_ROWS = _CB * _TN // 128
_NVR = _DIM // _L


def _make_sc_partials():
    mesh = plsc.VectorSubcoreMesh(core_axis_name="c", subcore_axis_name="s")

    @functools.partial(
        pl.kernel,
        mesh=mesh,
        compiler_params=pltpu.CompilerParams(use_tc_tiling_on_sc=False),
        out_type=jax.ShapeDtypeStruct((_B, _TN * _L), jnp.float32),
        scratch_types=[
            pltpu.VMEM((_CTX_IDX_ROWS, 128), jnp.int32),
            pltpu.VMEM((_TN_IDX_ROWS, 128), jnp.int32),
            pltpu.VMEM((_CB * _CTX, _DIM), jnp.float32),
            pltpu.VMEM((_CB * _TN, _DIM), jnp.float32),
            pltpu.VMEM((_CB, _TN * _L), jnp.float32),
            pltpu.SemaphoreType.DMA,
        ],
    )
    def sc_partials(ctx_idx_hbm, tn_idx_hbm, emb_hbm, cemb_hbm, out_hbm,
                    ctx_idx_v, tn_idx_v, crows_v, trows_v, out_v, sem):
        wid = lax.axis_index("s") * _NC + lax.axis_index("c")

        def chunk_body(g, carry):
            cbase = wid * _BPW + g * _CB
            gchunk = wid * _NCHUNK + g
            pltpu.sync_copy(ctx_idx_hbm.at[gchunk], ctx_idx_v)
            pltpu.sync_copy(tn_idx_hbm.at[gchunk], tn_idx_v)
            copies = []
            for j in range(_CTX_IDX_ROWS):
                copies.append(pltpu.async_copy(
                    cemb_hbm.at[ctx_idx_v.at[j]],
                    crows_v.at[pl.ds(j * 128, 128)], sem))
            for j in range(_TN_IDX_ROWS):
                copies.append(pltpu.async_copy(
                    emb_hbm.at[tn_idx_v.at[j]],
                    trows_v.at[pl.ds(j * 128, 128)], sem))
            for c in copies:
                c.wait()

            def b_body(b, carry2):
                rb = b * _CTX
                tb = b * _TN
                cs = [crows_v[rb, pl.ds(_L * j, _L)] for j in range(_NVR)]
                for c in range(1, _CTX):
                    for j in range(_NVR):
                        cs[j] = cs[j] + crows_v[rb + c, pl.ds(_L * j, _L)]
                scale = jnp.float32(1.0 / _CTX)
                cs = [v * scale for v in cs]
                for t in range(_TN):
                    p = cs[0] * trows_v[tb + t, pl.ds(0, _L)]
                    for j in range(1, _NVR):
                        p = p + cs[j] * trows_v[tb + t, pl.ds(_L * j, _L)]
                    out_v[b, pl.ds(_L * t, _L)] = p
                return carry2

            lax.fori_loop(0, _CB, b_body, 0)
            pltpu.sync_copy(out_v, out_hbm.at[pl.ds(cbase, _CB)])
            return carry

        lax.fori_loop(0, _NCHUNK, chunk_body, 0)

    return sc_partials


_RL_C = 1024
_RL_GRID = (_VOCAB + _RL_C - 1) // _RL_C


def _tc_relayout_body(in_ref, out_ref):
    x = in_ref[...]
    eye = (lax.broadcasted_iota(jnp.int32, (_DIM, _DIM), 0)
           == lax.broadcasted_iota(jnp.int32, (_DIM, _DIM), 1)
           ).astype(jnp.float32)
    t = lax.dot_general(x, eye, (((0,), (0,)), ((), ())),
                        preferred_element_type=jnp.float32)
    out_ref[...] = jnp.concatenate([t, t], axis=1)


def _tc_relayout(table):
    return pl.pallas_call(
        _tc_relayout_body,
        grid=(_RL_GRID,),
        in_specs=[pl.BlockSpec((_DIM, _RL_C), lambda i: (0, i))],
        out_specs=pl.BlockSpec((_RL_C, 2 * _DIM), lambda i: (i, 0)),
        out_shape=jax.ShapeDtypeStruct((_VOCAB, 2 * _DIM), jnp.float32),
    )(table.T)


_TC_BB = 2048


def _tc_loss_body(d_ref, out_ref):
    @pl.when(pl.program_id(0) == 0)
    def _init():
        out_ref[0, 0] = jnp.float32(0.0)

    x = d_ref[...]
    total = jnp.float32(0.0)
    for t in range(_TN):
        s = jnp.sum(x[:, _L * t:_L * (t + 1)], axis=1)
        ls = jnp.minimum(s, 0.0) - jnp.log(1.0 + jnp.exp(-jnp.abs(s)))
        total = total + jnp.sum(ls)
    out_ref[0, 0] = out_ref[0, 0] + total


def _tc_total(partials):
    return pl.pallas_call(
        _tc_loss_body,
        grid=(_B // _TC_BB,),
        in_specs=[pl.BlockSpec((_TC_BB, _TN * _L), lambda i: (i, 0))],
        out_specs=pl.BlockSpec(memory_space=pltpu.SMEM),
        out_shape=jax.ShapeDtypeStruct((1, 1), jnp.float32),
    )(partials)


def kernel(context, target, embeddings, context_embeddings):
    noise = jax.random.randint(jax.random.key(1), (target.shape[0], _NEG), 0,
                               _VOCAB)
    tn = jnp.concatenate([target[:, None], noise.astype(jnp.int32)], axis=1)
    ctx3d = (context * 2).reshape(_B // _CB, _CTX_IDX_ROWS, 128)
    tn3d = (tn * 2).reshape(_B // _CB, _TN_IDX_ROWS, 128)
    emb2 = _tc_relayout(embeddings).reshape(2 * _VOCAB, _DIM)
    cemb2 = _tc_relayout(context_embeddings).reshape(2 * _VOCAB, _DIM)
    partials = _make_sc_partials()(ctx3d, tn3d, emb2, cemb2)
    total = _tc_total(partials)
    return -(total[0, 0] / jnp.float32(_B))


# speedup vs baseline: 1.8421x; 1.8421x over previous
import functools

import jax
import jax.numpy as jnp
from jax import lax
from jax.experimental import pallas as pl
from jax.experimental.pallas import tpu as pltpu
from jax.experimental.pallas import tpu_sc as plsc

_VOCAB = 1000000
_DIM = 64
_NEG = 5
_B = 16384
_CTX = 20
_TN = _NEG + 1

_NC, _NS, _L = 2, 16, 16
_NW = _NC * _NS
_BPW = _B // _NW
_CB = 64
_NCHUNK = _BPW // _CB
_CTX_IDX_ROWS = _CB * _CTX // 128
_TN_IDX_ROWS = _CB * _TN // 128
_NVR = _DIM // _L


def _make_sc_partials():
    mesh = plsc.VectorSubcoreMesh(core_axis_name="c", subcore_axis_name="s")

    @functools.partial(
        pl.kernel,
        mesh=mesh,
        compiler_params=pltpu.CompilerParams(use_tc_tiling_on_sc=False),
        out_type=jax.ShapeDtypeStruct((_B, _TN * _L), jnp.float32),
        scratch_types=[
            pltpu.VMEM((_CTX_IDX_ROWS, 128), jnp.int32),
            pltpu.VMEM((_TN_IDX_ROWS, 128), jnp.int32),
            pltpu.VMEM((_CB * _CTX, _DIM), jnp.float32),
            pltpu.VMEM((_CB * _TN, _DIM), jnp.float32),
            pltpu.VMEM((_CB, _TN * _L), jnp.float32),
            pltpu.SemaphoreType.DMA,
        ],
    )
    def sc_partials(ctx_idx_hbm, tn_idx_hbm, emb_hbm, cemb_hbm, out_hbm,
                    ctx_idx_v, tn_idx_v, crows_v, trows_v, out_v, sem):
        wid = lax.axis_index("s") * _NC + lax.axis_index("c")

        def chunk_body(g, carry):
            cbase = wid * _BPW + g * _CB
            gchunk = wid * _NCHUNK + g
            pltpu.sync_copy(ctx_idx_hbm.at[gchunk], ctx_idx_v)
            pltpu.sync_copy(tn_idx_hbm.at[gchunk], tn_idx_v)
            copies = []
            for j in range(_CTX_IDX_ROWS):
                copies.append(pltpu.async_copy(
                    cemb_hbm.at[ctx_idx_v.at[j]],
                    crows_v.at[pl.ds(j * 128, 128)], sem))
            for j in range(_TN_IDX_ROWS):
                copies.append(pltpu.async_copy(
                    emb_hbm.at[tn_idx_v.at[j]],
                    trows_v.at[pl.ds(j * 128, 128)], sem))
            for c in copies:
                c.wait()

            def b_body(b, carry2):
                rb = b * _CTX
                tb = b * _TN
                cs = [crows_v[rb, pl.ds(_L * j, _L)] for j in range(_NVR)]
                for c in range(1, _CTX):
                    for j in range(_NVR):
                        cs[j] = cs[j] + crows_v[rb + c, pl.ds(_L * j, _L)]
                scale = jnp.float32(1.0 / _CTX)
                cs = [v * scale for v in cs]
                for t in range(_TN):
                    p = cs[0] * trows_v[tb + t, pl.ds(0, _L)]
                    for j in range(1, _NVR):
                        p = p + cs[j] * trows_v[tb + t, pl.ds(_L * j, _L)]
                    out_v[b, pl.ds(_L * t, _L)] = p
                return carry2

            lax.fori_loop(0, _CB, b_body, 0)
            pltpu.sync_copy(out_v, out_hbm.at[pl.ds(cbase, _CB)])
            return carry

        lax.fori_loop(0, _NCHUNK, chunk_body, 0)

    return sc_partials


_RL_C = 4096
_RL_GRID = (_VOCAB + _RL_C - 1) // _RL_C


def _tc_relayout_body(in_ref, out_ref):
    x = in_ref[...]
    eye = (lax.broadcasted_iota(jnp.int32, (_DIM, _DIM), 0)
           == lax.broadcasted_iota(jnp.int32, (_DIM, _DIM), 1)
           ).astype(jnp.float32)
    t = lax.dot_general(x, eye, (((0,), (0,)), ((), ())),
                        preferred_element_type=jnp.float32)
    out_ref[...] = jnp.concatenate([t, t], axis=1)


def _tc_relayout(table):
    return pl.pallas_call(
        _tc_relayout_body,
        grid=(_RL_GRID,),
        in_specs=[pl.BlockSpec((_DIM, _RL_C), lambda i: (0, i))],
        out_specs=pl.BlockSpec((_RL_C, 2 * _DIM), lambda i: (i, 0)),
        out_shape=jax.ShapeDtypeStruct((_VOCAB, 2 * _DIM), jnp.float32),
    )(table.T)


_TC_BB = 2048


def _tc_loss_body(d_ref, out_ref):
    @pl.when(pl.program_id(0) == 0)
    def _init():
        out_ref[0, 0] = jnp.float32(0.0)

    x = d_ref[...]
    total = jnp.float32(0.0)
    for t in range(_TN):
        s = jnp.sum(x[:, _L * t:_L * (t + 1)], axis=1)
        ls = jnp.minimum(s, 0.0) - jnp.log(1.0 + jnp.exp(-jnp.abs(s)))
        total = total + jnp.sum(ls)
    out_ref[0, 0] = out_ref[0, 0] + total


def _tc_total(partials):
    return pl.pallas_call(
        _tc_loss_body,
        grid=(_B // _TC_BB,),
        in_specs=[pl.BlockSpec((_TC_BB, _TN * _L), lambda i: (i, 0))],
        out_specs=pl.BlockSpec(memory_space=pltpu.SMEM),
        out_shape=jax.ShapeDtypeStruct((1, 1), jnp.float32),
    )(partials)


def kernel(context, target, embeddings, context_embeddings):
    noise = jax.random.randint(jax.random.key(1), (target.shape[0], _NEG), 0,
                               _VOCAB)
    tn = jnp.concatenate([target[:, None], noise.astype(jnp.int32)], axis=1)
    ctx3d = (context * 2).reshape(_B // _CB, _CTX_IDX_ROWS, 128)
    tn3d = (tn * 2).reshape(_B // _CB, _TN_IDX_ROWS, 128)
    emb2 = _tc_relayout(embeddings).reshape(2 * _VOCAB, _DIM)
    cemb2 = _tc_relayout(context_embeddings).reshape(2 * _VOCAB, _DIM)
    partials = _make_sc_partials()(ctx3d, tn3d, emb2, cemb2)
    total = _tc_total(partials)
    return -(total[0, 0] / jnp.float32(_B))


# speedup vs baseline: 2.1510x; 1.1677x over previous
import functools

import jax
import jax.numpy as jnp
from jax import lax
from jax.experimental import pallas as pl
from jax.experimental.pallas import tpu as pltpu
from jax.experimental.pallas import tpu_sc as plsc

_VOCAB = 1000000
_DIM = 64
_NEG = 5
_B = 16384
_CTX = 20
_TN = _NEG + 1

_NC, _NS, _L = 2, 16, 16
_NW = _NC * _NS
_BPW = _B // _NW
_CB = 64
_NCHUNK = _BPW // _CB
_CTX_IDX_ROWS = _CB * _CTX // 128
_TN_IDX_ROWS = _CB * _TN // 128
_NVR = _DIM // _L


def _make_sc_partials():
    mesh = plsc.VectorSubcoreMesh(core_axis_name="c", subcore_axis_name="s")

    @functools.partial(
        pl.kernel,
        mesh=mesh,
        compiler_params=pltpu.CompilerParams(use_tc_tiling_on_sc=False),
        out_type=jax.ShapeDtypeStruct((_B, _TN * _L), jnp.float32),
        scratch_types=[
            pltpu.VMEM((_CTX_IDX_ROWS, 128), jnp.int32),
            pltpu.VMEM((_TN_IDX_ROWS, 128), jnp.int32),
            pltpu.VMEM((_CB * _CTX, _DIM), jnp.float32),
            pltpu.VMEM((_CB * _TN, _DIM), jnp.float32),
            pltpu.VMEM((_CB, _TN * _L), jnp.float32),
            pltpu.SemaphoreType.DMA,
        ],
    )
    def sc_partials(ctx_idx_hbm, tn_idx_hbm, emb_hbm, cemb_hbm, out_hbm,
                    ctx_idx_v, tn_idx_v, crows_v, trows_v, out_v, sem):
        wid = lax.axis_index("s") * _NC + lax.axis_index("c")

        def chunk_body(g, carry):
            cbase = wid * _BPW + g * _CB
            gchunk = wid * _NCHUNK + g
            pltpu.sync_copy(ctx_idx_hbm.at[gchunk], ctx_idx_v)
            pltpu.sync_copy(tn_idx_hbm.at[gchunk], tn_idx_v)
            copies = []
            for j in range(_CTX_IDX_ROWS):
                copies.append(pltpu.async_copy(
                    cemb_hbm.at[ctx_idx_v.at[j]],
                    crows_v.at[pl.ds(j * 128, 128)], sem))
            for j in range(_TN_IDX_ROWS):
                copies.append(pltpu.async_copy(
                    emb_hbm.at[tn_idx_v.at[j]],
                    trows_v.at[pl.ds(j * 128, 128)], sem))
            for c in copies:
                c.wait()

            def b_body(b, carry2):
                rb = b * _CTX
                tb = b * _TN
                cs = [crows_v[rb, pl.ds(_L * j, _L)] for j in range(_NVR)]
                for c in range(1, _CTX):
                    for j in range(_NVR):
                        cs[j] = cs[j] + crows_v[rb + c, pl.ds(_L * j, _L)]
                scale = jnp.float32(1.0 / _CTX)
                cs = [v * scale for v in cs]
                for t in range(_TN):
                    p = cs[0] * trows_v[tb + t, pl.ds(0, _L)]
                    for j in range(1, _NVR):
                        p = p + cs[j] * trows_v[tb + t, pl.ds(_L * j, _L)]
                    out_v[b, pl.ds(_L * t, _L)] = p
                return carry2

            lax.fori_loop(0, _CB, b_body, 0)
            pltpu.sync_copy(out_v, out_hbm.at[pl.ds(cbase, _CB)])
            return carry

        lax.fori_loop(0, _NCHUNK, chunk_body, 0)

    return sc_partials


_RL_C = 8192
_RL_GRID = (_VOCAB + _RL_C - 1) // _RL_C


def _tc_relayout_body(in_ref, out_ref):
    x = in_ref[...]
    eye = (lax.broadcasted_iota(jnp.int32, (_DIM, _DIM), 0)
           == lax.broadcasted_iota(jnp.int32, (_DIM, _DIM), 1)
           ).astype(jnp.float32)
    t = lax.dot_general(x, eye, (((0,), (0,)), ((), ())),
                        preferred_element_type=jnp.float32)
    out_ref[...] = jnp.concatenate([t, t], axis=1)


def _tc_relayout(table):
    return pl.pallas_call(
        _tc_relayout_body,
        grid=(_RL_GRID,),
        in_specs=[pl.BlockSpec((_DIM, _RL_C), lambda i: (0, i))],
        out_specs=pl.BlockSpec((_RL_C, 2 * _DIM), lambda i: (i, 0)),
        out_shape=jax.ShapeDtypeStruct((_VOCAB, 2 * _DIM), jnp.float32),
    )(table.T)


_TC_BB = 2048


def _tc_loss_body(d_ref, out_ref):
    @pl.when(pl.program_id(0) == 0)
    def _init():
        out_ref[0, 0] = jnp.float32(0.0)

    x = d_ref[...]
    total = jnp.float32(0.0)
    for t in range(_TN):
        s = jnp.sum(x[:, _L * t:_L * (t + 1)], axis=1)
        ls = jnp.minimum(s, 0.0) - jnp.log(1.0 + jnp.exp(-jnp.abs(s)))
        total = total + jnp.sum(ls)
    out_ref[0, 0] = out_ref[0, 0] + total


def _tc_total(partials):
    return pl.pallas_call(
        _tc_loss_body,
        grid=(_B // _TC_BB,),
        in_specs=[pl.BlockSpec((_TC_BB, _TN * _L), lambda i: (i, 0))],
        out_specs=pl.BlockSpec(memory_space=pltpu.SMEM),
        out_shape=jax.ShapeDtypeStruct((1, 1), jnp.float32),
    )(partials)


def kernel(context, target, embeddings, context_embeddings):
    noise = jax.random.randint(jax.random.key(1), (target.shape[0], _NEG), 0,
                               _VOCAB)
    tn = jnp.concatenate([target[:, None], noise.astype(jnp.int32)], axis=1)
    ctx3d = (context * 2).reshape(_B // _CB, _CTX_IDX_ROWS, 128)
    tn3d = (tn * 2).reshape(_B // _CB, _TN_IDX_ROWS, 128)
    emb2 = _tc_relayout(embeddings).reshape(2 * _VOCAB, _DIM)
    cemb2 = _tc_relayout(context_embeddings).reshape(2 * _VOCAB, _DIM)
    partials = _make_sc_partials()(ctx3d, tn3d, emb2, cemb2)
    total = _tc_total(partials)
    return -(total[0, 0] / jnp.float32(_B))


# speedup vs baseline: 2.3450x; 1.0902x over previous
import functools

import jax
import jax.numpy as jnp
from jax import lax
from jax.experimental import pallas as pl
from jax.experimental.pallas import tpu as pltpu
from jax.experimental.pallas import tpu_sc as plsc

_VOCAB = 1000000
_DIM = 64
_NEG = 5
_B = 16384
_CTX = 20
_TN = _NEG + 1

_NC, _NS, _L = 2, 16, 16
_NW = _NC * _NS
_BPW = _B // _NW
_CB = 64
_NCHUNK = _BPW // _CB
_CTX_IDX_ROWS = _CB * _CTX // 128
_TN_IDX_ROWS = _CB * _TN // 128
_NVR = _DIM // _L


def _make_sc_partials():
    mesh = plsc.VectorSubcoreMesh(core_axis_name="c", subcore_axis_name="s")

    @functools.partial(
        pl.kernel,
        mesh=mesh,
        compiler_params=pltpu.CompilerParams(use_tc_tiling_on_sc=False),
        out_type=jax.ShapeDtypeStruct((_B, _TN * _L), jnp.float32),
        scratch_types=[
            pltpu.VMEM((_CTX_IDX_ROWS, 128), jnp.int32),
            pltpu.VMEM((_TN_IDX_ROWS, 128), jnp.int32),
            pltpu.VMEM((_CB * _CTX, _DIM), jnp.float32),
            pltpu.VMEM((_CB * _TN, _DIM), jnp.float32),
            pltpu.VMEM((_CB, _TN * _L), jnp.float32),
            pltpu.SemaphoreType.DMA,
        ],
    )
    def sc_partials(ctx_idx_hbm, tn_idx_hbm, emb_hbm, cemb_hbm, out_hbm,
                    ctx_idx_v, tn_idx_v, crows_v, trows_v, out_v, sem):
        wid = lax.axis_index("s") * _NC + lax.axis_index("c")

        def chunk_body(g, carry):
            cbase = wid * _BPW + g * _CB
            gchunk = wid * _NCHUNK + g
            pltpu.sync_copy(ctx_idx_hbm.at[gchunk], ctx_idx_v)
            pltpu.sync_copy(tn_idx_hbm.at[gchunk], tn_idx_v)
            copies = []
            for j in range(_CTX_IDX_ROWS):
                copies.append(pltpu.async_copy(
                    cemb_hbm.at[ctx_idx_v.at[j]],
                    crows_v.at[pl.ds(j * 128, 128)], sem))
            for j in range(_TN_IDX_ROWS):
                copies.append(pltpu.async_copy(
                    emb_hbm.at[tn_idx_v.at[j]],
                    trows_v.at[pl.ds(j * 128, 128)], sem))
            for c in copies:
                c.wait()

            def b_body(b, carry2):
                rb = b * _CTX
                tb = b * _TN
                cs = [crows_v[rb, pl.ds(_L * j, _L)] for j in range(_NVR)]
                for c in range(1, _CTX):
                    for j in range(_NVR):
                        cs[j] = cs[j] + crows_v[rb + c, pl.ds(_L * j, _L)]
                scale = jnp.float32(1.0 / _CTX)
                cs = [v * scale for v in cs]
                for t in range(_TN):
                    p = cs[0] * trows_v[tb + t, pl.ds(0, _L)]
                    for j in range(1, _NVR):
                        p = p + cs[j] * trows_v[tb + t, pl.ds(_L * j, _L)]
                    out_v[b, pl.ds(_L * t, _L)] = p
                return carry2

            lax.fori_loop(0, _CB, b_body, 0)
            pltpu.sync_copy(out_v, out_hbm.at[pl.ds(cbase, _CB)])
            return carry

        lax.fori_loop(0, _NCHUNK, chunk_body, 0)

    return sc_partials


_RL_C = 16384
_RL_GRID = (_VOCAB + _RL_C - 1) // _RL_C


def _tc_relayout_body(in_ref, out_ref):
    x = in_ref[...]
    eye = (lax.broadcasted_iota(jnp.int32, (_DIM, _DIM), 0)
           == lax.broadcasted_iota(jnp.int32, (_DIM, _DIM), 1)
           ).astype(jnp.float32)
    t = lax.dot_general(x, eye, (((0,), (0,)), ((), ())),
                        preferred_element_type=jnp.float32)
    out_ref[...] = jnp.concatenate([t, t], axis=1)


def _tc_relayout(table):
    return pl.pallas_call(
        _tc_relayout_body,
        grid=(_RL_GRID,),
        in_specs=[pl.BlockSpec((_DIM, _RL_C), lambda i: (0, i))],
        out_specs=pl.BlockSpec((_RL_C, 2 * _DIM), lambda i: (i, 0)),
        out_shape=jax.ShapeDtypeStruct((_VOCAB, 2 * _DIM), jnp.float32),
    )(table.T)


_TC_BB = 2048


def _tc_loss_body(d_ref, out_ref):
    @pl.when(pl.program_id(0) == 0)
    def _init():
        out_ref[0, 0] = jnp.float32(0.0)

    x = d_ref[...]
    total = jnp.float32(0.0)
    for t in range(_TN):
        s = jnp.sum(x[:, _L * t:_L * (t + 1)], axis=1)
        ls = jnp.minimum(s, 0.0) - jnp.log(1.0 + jnp.exp(-jnp.abs(s)))
        total = total + jnp.sum(ls)
    out_ref[0, 0] = out_ref[0, 0] + total


def _tc_total(partials):
    return pl.pallas_call(
        _tc_loss_body,
        grid=(_B // _TC_BB,),
        in_specs=[pl.BlockSpec((_TC_BB, _TN * _L), lambda i: (i, 0))],
        out_specs=pl.BlockSpec(memory_space=pltpu.SMEM),
        out_shape=jax.ShapeDtypeStruct((1, 1), jnp.float32),
    )(partials)


def kernel(context, target, embeddings, context_embeddings):
    noise = jax.random.randint(jax.random.key(1), (target.shape[0], _NEG), 0,
                               _VOCAB)
    tn = jnp.concatenate([target[:, None], noise.astype(jnp.int32)], axis=1)
    ctx3d = (context * 2).reshape(_B // _CB, _CTX_IDX_ROWS, 128)
    tn3d = (tn * 2).reshape(_B // _CB, _TN_IDX_ROWS, 128)
    emb2 = _tc_relayout(embeddings).reshape(2 * _VOCAB, _DIM)
    cemb2 = _tc_relayout(context_embeddings).reshape(2 * _VOCAB, _DIM)
    partials = _make_sc_partials()(ctx3d, tn3d, emb2, cemb2)
    total = _tc_total(partials)
    return -(total[0, 0] / jnp.float32(_B))


# speedup vs baseline: 2.4595x; 1.0488x over previous
import functools

import jax
import jax.numpy as jnp
from jax import lax
from jax.experimental import pallas as pl
from jax.experimental.pallas import tpu as pltpu
from jax.experimental.pallas import tpu_sc as plsc

_VOCAB = 1000000
_DIM = 64
_NEG = 5
_B = 16384
_CTX = 20
_TN = _NEG + 1

_NC, _NS, _L = 2, 16, 16
_NW = _NC * _NS
_BPW = _B // _NW
_CB = 64
_NCHUNK = _BPW // _CB
_CTX_IDX_ROWS = _CB * _CTX // 128
_TN_IDX_ROWS = _CB * _TN // 128
_NVR = _DIM // _L


def _make_sc_pool():
    mesh = plsc.VectorSubcoreMesh(core_axis_name="c", subcore_axis_name="s")

    @functools.partial(
        pl.kernel,
        mesh=mesh,
        compiler_params=pltpu.CompilerParams(use_tc_tiling_on_sc=False),
        out_type=jax.ShapeDtypeStruct((_B, _DIM), jnp.float32),
        scratch_types=[
            pltpu.VMEM((_CTX_IDX_ROWS, 128), jnp.int32),
            pltpu.VMEM((_CB * _CTX, _DIM), jnp.float32),
            pltpu.VMEM((_CB, _DIM), jnp.float32),
            pltpu.SemaphoreType.DMA,
        ],
    )
    def sc_pool(ctx_idx_hbm, cemb_hbm, out_hbm, ctx_idx_v, crows_v, pool_v,
                sem):
        wid = lax.axis_index("s") * _NC + lax.axis_index("c")

        def chunk_body(g, carry):
            cbase = wid * _BPW + g * _CB
            gchunk = wid * _NCHUNK + g
            pltpu.sync_copy(ctx_idx_hbm.at[gchunk], ctx_idx_v)
            copies = []
            for j in range(_CTX_IDX_ROWS):
                copies.append(pltpu.async_copy(
                    cemb_hbm.at[ctx_idx_v.at[j]],
                    crows_v.at[pl.ds(j * 128, 128)], sem))
            for c in copies:
                c.wait()

            def b_body(b, carry2):
                rb = b * _CTX
                scale = jnp.float32(1.0 / _CTX)
                for j in range(_NVR):
                    s = crows_v[rb, pl.ds(_L * j, _L)]
                    for c in range(1, _CTX):
                        s = s + crows_v[rb + c, pl.ds(_L * j, _L)]
                    pool_v[b, pl.ds(_L * j, _L)] = s * scale
                return carry2

            lax.fori_loop(0, _CB, b_body, 0)
            pltpu.sync_copy(pool_v, out_hbm.at[pl.ds(cbase, _CB)])
            return carry

        lax.fori_loop(0, _NCHUNK, chunk_body, 0)

    return sc_pool


def _make_sc_dots():
    mesh = plsc.VectorSubcoreMesh(core_axis_name="c", subcore_axis_name="s")

    @functools.partial(
        pl.kernel,
        mesh=mesh,
        compiler_params=pltpu.CompilerParams(use_tc_tiling_on_sc=False),
        out_type=jax.ShapeDtypeStruct((_B, _TN * _L), jnp.float32),
        scratch_types=[
            pltpu.VMEM((_TN_IDX_ROWS, 128), jnp.int32),
            pltpu.VMEM((_CB * _TN, _DIM), jnp.float32),
            pltpu.VMEM((_CB, _DIM), jnp.float32),
            pltpu.VMEM((_CB, _TN * _L), jnp.float32),
            pltpu.SemaphoreType.DMA,
        ],
    )
    def sc_dots(tn_idx_hbm, emb_hbm, pooled_hbm, out_hbm, tn_idx_v, trows_v,
                pool_v, out_v, sem):
        wid = lax.axis_index("s") * _NC + lax.axis_index("c")

        def chunk_body(g, carry):
            cbase = wid * _BPW + g * _CB
            gchunk = wid * _NCHUNK + g
            pltpu.sync_copy(tn_idx_hbm.at[gchunk], tn_idx_v)
            copies = [pltpu.async_copy(pooled_hbm.at[pl.ds(cbase, _CB)],
                                       pool_v, sem)]
            for j in range(_TN_IDX_ROWS):
                copies.append(pltpu.async_copy(
                    emb_hbm.at[tn_idx_v.at[j]],
                    trows_v.at[pl.ds(j * 128, 128)], sem))
            for c in copies:
                c.wait()

            def b_body(b, carry2):
                tb = b * _TN
                cs = [pool_v[b, pl.ds(_L * j, _L)] for j in range(_NVR)]
                for t in range(_TN):
                    p = cs[0] * trows_v[tb + t, pl.ds(0, _L)]
                    for j in range(1, _NVR):
                        p = p + cs[j] * trows_v[tb + t, pl.ds(_L * j, _L)]
                    out_v[b, pl.ds(_L * t, _L)] = p
                return carry2

            lax.fori_loop(0, _CB, b_body, 0)
            pltpu.sync_copy(out_v, out_hbm.at[pl.ds(cbase, _CB)])
            return carry

        lax.fori_loop(0, _NCHUNK, chunk_body, 0)

    return sc_dots


_RL_C = 16384
_RL_GRID = (_VOCAB + _RL_C - 1) // _RL_C


def _tc_relayout_body(in_ref, out_ref):
    x = in_ref[...]
    eye = (lax.broadcasted_iota(jnp.int32, (_DIM, _DIM), 0)
           == lax.broadcasted_iota(jnp.int32, (_DIM, _DIM), 1)
           ).astype(jnp.float32)
    t = lax.dot_general(x, eye, (((0,), (0,)), ((), ())),
                        preferred_element_type=jnp.float32)
    out_ref[...] = jnp.concatenate([t, t], axis=1)


def _tc_relayout(table):
    return pl.pallas_call(
        _tc_relayout_body,
        grid=(_RL_GRID,),
        in_specs=[pl.BlockSpec((_DIM, _RL_C), lambda i: (0, i))],
        out_specs=pl.BlockSpec((_RL_C, 2 * _DIM), lambda i: (i, 0)),
        out_shape=jax.ShapeDtypeStruct((_VOCAB, 2 * _DIM), jnp.float32),
    )(table.T)


_TC_BB = 2048


def _tc_loss_body(d_ref, out_ref):
    @pl.when(pl.program_id(0) == 0)
    def _init():
        out_ref[0, 0] = jnp.float32(0.0)

    x = d_ref[...]
    total = jnp.float32(0.0)
    for t in range(_TN):
        s = jnp.sum(x[:, _L * t:_L * (t + 1)], axis=1)
        ls = jnp.minimum(s, 0.0) - jnp.log(1.0 + jnp.exp(-jnp.abs(s)))
        total = total + jnp.sum(ls)
    out_ref[0, 0] = out_ref[0, 0] + total


def _tc_total(partials):
    return pl.pallas_call(
        _tc_loss_body,
        grid=(_B // _TC_BB,),
        in_specs=[pl.BlockSpec((_TC_BB, _TN * _L), lambda i: (i, 0))],
        out_specs=pl.BlockSpec(memory_space=pltpu.SMEM),
        out_shape=jax.ShapeDtypeStruct((1, 1), jnp.float32),
    )(partials)


def kernel(context, target, embeddings, context_embeddings):
    noise = jax.random.randint(jax.random.key(1), (target.shape[0], _NEG), 0,
                               _VOCAB)
    tn = jnp.concatenate([target[:, None], noise.astype(jnp.int32)], axis=1)
    ctx3d = (context * 2).reshape(_B // _CB, _CTX_IDX_ROWS, 128)
    tn3d = (tn * 2).reshape(_B // _CB, _TN_IDX_ROWS, 128)
    cemb2 = _tc_relayout(context_embeddings).reshape(2 * _VOCAB, _DIM)
    pooled = _make_sc_pool()(ctx3d, cemb2)
    emb2 = _tc_relayout(embeddings).reshape(2 * _VOCAB, _DIM)
    partials = _make_sc_dots()(tn3d, emb2, pooled)
    total = _tc_total(partials)
    return -(total[0, 0] / jnp.float32(_B))


# speedup vs baseline: 2.9830x; 1.2129x over previous
import functools

import jax
import jax.numpy as jnp
from jax import lax
from jax.experimental import pallas as pl
from jax.experimental.pallas import tpu as pltpu
from jax.experimental.pallas import tpu_sc as plsc

_VOCAB = 1000000
_DIM = 64
_NEG = 5
_B = 16384
_CTX = 20
_TN = _NEG + 1

_NC, _NS, _L = 2, 16, 16
_NW = _NC * _NS
_BPW = _B // _NW
_CB = 64
_NCHUNK = _BPW // _CB
_CTX_IDX_ROWS = _CB * _CTX // 128
_TN_IDX_ROWS = _CB * _TN // 128
_NVR = _DIM // _L


def _make_sc_pool():
    mesh = plsc.VectorSubcoreMesh(core_axis_name="c", subcore_axis_name="s")

    @functools.partial(
        pl.kernel,
        mesh=mesh,
        compiler_params=pltpu.CompilerParams(use_tc_tiling_on_sc=False),
        out_type=jax.ShapeDtypeStruct((_B, _DIM), jnp.float32),
        scratch_types=[
            pltpu.VMEM((_CTX_IDX_ROWS, 128), jnp.int32),
            pltpu.VMEM((_CB * _CTX, _DIM), jnp.float32),
            pltpu.VMEM((_CB, _DIM), jnp.float32),
            pltpu.SemaphoreType.DMA,
        ],
    )
    def sc_pool(ctx_idx_hbm, cemb_hbm, out_hbm, ctx_idx_v, crows_v, pool_v,
                sem):
        wid = lax.axis_index("s") * _NC + lax.axis_index("c")

        def chunk_body(g, carry):
            cbase = wid * _BPW + g * _CB
            gchunk = wid * _NCHUNK + g
            pltpu.sync_copy(ctx_idx_hbm.at[gchunk], ctx_idx_v)
            copies = []
            for j in range(_CTX_IDX_ROWS):
                copies.append(pltpu.async_copy(
                    cemb_hbm.at[ctx_idx_v.at[j]],
                    crows_v.at[pl.ds(j * 128, 128)], sem))
            for c in copies:
                c.wait()

            def b_body(b, carry2):
                rb = b * _CTX
                scale = jnp.float32(1.0 / _CTX)
                for j in range(_NVR):
                    s = crows_v[rb, pl.ds(_L * j, _L)]
                    for c in range(1, _CTX):
                        s = s + crows_v[rb + c, pl.ds(_L * j, _L)]
                    pool_v[b, pl.ds(_L * j, _L)] = s * scale
                return carry2

            lax.fori_loop(0, _CB, b_body, 0)
            pltpu.sync_copy(pool_v, out_hbm.at[pl.ds(cbase, _CB)])
            return carry

        lax.fori_loop(0, _NCHUNK, chunk_body, 0)

    return sc_pool


def _make_sc_dots():
    mesh = plsc.VectorSubcoreMesh(core_axis_name="c", subcore_axis_name="s")

    @functools.partial(
        pl.kernel,
        mesh=mesh,
        compiler_params=pltpu.CompilerParams(use_tc_tiling_on_sc=False),
        out_type=jax.ShapeDtypeStruct((_B, _TN * _L), jnp.float32),
        scratch_types=[
            pltpu.VMEM((_TN_IDX_ROWS, 128), jnp.int32),
            pltpu.VMEM((_CB * _TN, _DIM), jnp.float32),
            pltpu.VMEM((_CB, _DIM), jnp.float32),
            pltpu.VMEM((_CB, _TN * _L), jnp.float32),
            pltpu.SemaphoreType.DMA,
        ],
    )
    def sc_dots(tn_idx_hbm, emb_hbm, pooled_hbm, out_hbm, tn_idx_v, trows_v,
                pool_v, out_v, sem):
        wid = lax.axis_index("s") * _NC + lax.axis_index("c")

        def chunk_body(g, carry):
            cbase = wid * _BPW + g * _CB
            gchunk = wid * _NCHUNK + g
            pltpu.sync_copy(tn_idx_hbm.at[gchunk], tn_idx_v)
            copies = [pltpu.async_copy(pooled_hbm.at[pl.ds(cbase, _CB)],
                                       pool_v, sem)]
            for j in range(_TN_IDX_ROWS):
                copies.append(pltpu.async_copy(
                    emb_hbm.at[tn_idx_v.at[j]],
                    trows_v.at[pl.ds(j * 128, 128)], sem))
            for c in copies:
                c.wait()

            def b_body(b, carry2):
                tb = b * _TN
                cs = [pool_v[b, pl.ds(_L * j, _L)] for j in range(_NVR)]
                for t in range(_TN):
                    p = cs[0] * trows_v[tb + t, pl.ds(0, _L)]
                    for j in range(1, _NVR):
                        p = p + cs[j] * trows_v[tb + t, pl.ds(_L * j, _L)]
                    out_v[b, pl.ds(_L * t, _L)] = p
                return carry2

            lax.fori_loop(0, _CB, b_body, 0)
            pltpu.sync_copy(out_v, out_hbm.at[pl.ds(cbase, _CB)])
            return carry

        lax.fori_loop(0, _NCHUNK, chunk_body, 0)

    return sc_dots


_RL_C = 16384
_RL_H = _RL_C // 2
_RL_GRID = (_VOCAB + _RL_C - 1) // _RL_C
_VP = _RL_GRID * _RL_C


def _tc_relayout_body(in_ref, out_ref):
    x = in_ref[...]
    eye = (lax.broadcasted_iota(jnp.int32, (_DIM, _DIM), 0)
           == lax.broadcasted_iota(jnp.int32, (_DIM, _DIM), 1)
           ).astype(jnp.float32)
    t = lax.dot_general(x, eye, (((0,), (0,)), ((), ())),
                        preferred_element_type=jnp.float32)
    out_ref[...] = jnp.concatenate([t[:_RL_H], t[_RL_H:]], axis=1)


def _tc_relayout(table):
    return pl.pallas_call(
        _tc_relayout_body,
        grid=(_RL_GRID,),
        in_specs=[pl.BlockSpec((_DIM, _RL_C), lambda i: (0, i))],
        out_specs=pl.BlockSpec((_RL_H, 2 * _DIM), lambda i: (i, 0)),
        out_shape=jax.ShapeDtypeStruct((_VP // 2, 2 * _DIM), jnp.float32),
    )(table.T)


def _remap(idx):
    l = idx % _RL_C
    return idx + l + jnp.where(l < _RL_H, 0, 1 - _RL_C)


_TC_BB = 2048


def _tc_loss_body(d_ref, out_ref):
    @pl.when(pl.program_id(0) == 0)
    def _init():
        out_ref[0, 0] = jnp.float32(0.0)

    x = d_ref[...]
    total = jnp.float32(0.0)
    for t in range(_TN):
        s = jnp.sum(x[:, _L * t:_L * (t + 1)], axis=1)
        ls = jnp.minimum(s, 0.0) - jnp.log(1.0 + jnp.exp(-jnp.abs(s)))
        total = total + jnp.sum(ls)
    out_ref[0, 0] = out_ref[0, 0] + total


def _tc_total(partials):
    return pl.pallas_call(
        _tc_loss_body,
        grid=(_B // _TC_BB,),
        in_specs=[pl.BlockSpec((_TC_BB, _TN * _L), lambda i: (i, 0))],
        out_specs=pl.BlockSpec(memory_space=pltpu.SMEM),
        out_shape=jax.ShapeDtypeStruct((1, 1), jnp.float32),
    )(partials)


def kernel(context, target, embeddings, context_embeddings):
    noise = jax.random.randint(jax.random.key(1), (target.shape[0], _NEG), 0,
                               _VOCAB)
    tn = jnp.concatenate([target[:, None], noise.astype(jnp.int32)], axis=1)
    ctx3d = _remap(context).reshape(_B // _CB, _CTX_IDX_ROWS, 128)
    tn3d = _remap(tn).reshape(_B // _CB, _TN_IDX_ROWS, 128)
    cemb2 = _tc_relayout(context_embeddings).reshape(_VP, _DIM)
    pooled = _make_sc_pool()(ctx3d, cemb2)
    emb2 = _tc_relayout(embeddings).reshape(_VP, _DIM)
    partials = _make_sc_dots()(tn3d, emb2, pooled)
    total = _tc_total(partials)
    return -(total[0, 0] / jnp.float32(_B))
